# Initial kernel scaffold; baseline (speedup 1.0000x reference)
import functools

import jax
import jax.numpy as jnp
from jax import lax
from jax.experimental import pallas as pl
from jax.experimental.pallas import tpu as pltpu
from jax.experimental.pallas import tpu_sc as plsc


def _make_gather(V, D, N):
    info = plsc.get_sparse_core_info()
    NC, NS = info.num_cores, info.num_subcores
    NW = NC * NS
    assert N % NW == 0
    b_per_w = N // NW
    C = 1600
    assert b_per_w % C == 0
    n_chunks = b_per_w // C

    mesh = plsc.VectorSubcoreMesh(core_axis_name="c", subcore_axis_name="s")

    @functools.partial(
        pl.kernel,
        mesh=mesh,
        out_type=jax.ShapeDtypeStruct((N, D), jnp.float32),
        scratch_types=[
            pltpu.VMEM((C,), jnp.int32),
            pltpu.VMEM((C, D), jnp.float32),
            pltpu.SemaphoreType.DMA,
        ],
    )
    def gather_kernel(table_hbm, idx_hbm, out_hbm, idx_v, rows_v, sem):
        wid = lax.axis_index("s") * NC + lax.axis_index("c")
        base = wid * b_per_w

        def chunk_body(i, carry):
            off = base + i * C
            pltpu.sync_copy(idx_hbm.at[pl.ds(off, C)], idx_v)
            pltpu.async_copy(table_hbm.at[idx_v], rows_v, sem).wait()
            pltpu.sync_copy(rows_v, out_hbm.at[pl.ds(off, C)])
            return carry

        lax.fori_loop(0, n_chunks, chunk_body, 0)

    return gather_kernel


def kernel(inputs, table):
    B, S = inputs.shape
    V, D = table.shape
    idx = inputs.reshape(-1).astype(jnp.int32)
    gather = _make_gather(V, D, B * S)
    out = gather(table, idx)
    return out.reshape(B, S, D)


# speedup vs baseline: 3.2975x; 3.2975x over previous
import functools

import jax
import jax.numpy as jnp
from jax import lax
from jax.experimental import pallas as pl
from jax.experimental.pallas import tpu as pltpu
from jax.experimental.pallas import tpu_sc as plsc


def _make_gather(V, D, N):
    info = plsc.get_sparse_core_info()
    NC, NS = info.num_cores, info.num_subcores
    NW = NC * NS
    assert N % NW == 0
    b_per_w = N // NW
    C = 1600
    assert b_per_w % C == 0
    n_chunks = b_per_w // C

    mesh = plsc.VectorSubcoreMesh(core_axis_name="c", subcore_axis_name="s")

    @functools.partial(
        pl.kernel,
        mesh=mesh,
        out_type=jax.ShapeDtypeStruct((N, D), jnp.float32),
        compiler_params=pltpu.CompilerParams(use_tc_tiling_on_sc=False),
        scratch_types=[
            pltpu.VMEM((C,), jnp.int32),
            pltpu.VMEM((C, D), jnp.float32),
            pltpu.SemaphoreType.DMA,
        ],
    )
    def gather_kernel(table_hbm, idx_hbm, out_hbm, idx_v, rows_v, sem):
        wid = lax.axis_index("s") * NC + lax.axis_index("c")
        base = wid * b_per_w

        def chunk_body(i, carry):
            off = base + i * C
            pltpu.sync_copy(idx_hbm.at[pl.ds(off, C)], idx_v)
            pltpu.async_copy(table_hbm.at[idx_v], rows_v, sem).wait()
            pltpu.sync_copy(rows_v, out_hbm.at[pl.ds(off, C)])
            return carry

        lax.fori_loop(0, n_chunks, chunk_body, 0)

    return gather_kernel


def kernel(inputs, table):
    B, S = inputs.shape
    V, D = table.shape
    idx = inputs.reshape(-1).astype(jnp.int32)
    gather = _make_gather(V, D, B * S)
    out = gather(table, idx)
    return out.reshape(B, S, D)


# speedup vs baseline: 3.3090x; 1.0035x over previous
import functools

import jax
import jax.numpy as jnp
from jax import lax
from jax.experimental import pallas as pl
from jax.experimental.pallas import tpu as pltpu
from jax.experimental.pallas import tpu_sc as plsc

_NBUF = 2
_CHUNK = 800


def _make_gather(V, D, N):
    info = plsc.get_sparse_core_info()
    NC, NS = info.num_cores, info.num_subcores
    NW = NC * NS
    assert N % NW == 0
    b_per_w = N // NW
    C = _CHUNK
    assert b_per_w % (C * _NBUF) == 0
    n_chunks = b_per_w // C
    n_groups = n_chunks // _NBUF

    mesh = plsc.VectorSubcoreMesh(core_axis_name="c", subcore_axis_name="s")

    @functools.partial(
        pl.kernel,
        mesh=mesh,
        out_type=jax.ShapeDtypeStruct((N, D), jnp.float32),
        compiler_params=pltpu.CompilerParams(use_tc_tiling_on_sc=False),
        scratch_types=[
            pltpu.VMEM((b_per_w,), jnp.int32),
            pltpu.VMEM((_NBUF, C, D), jnp.float32),
            pltpu.SemaphoreType.DMA,
            pltpu.SemaphoreType.DMA,
        ],
    )
    def gather_kernel(table_hbm, idx_hbm, out_hbm, idx_v, rows_v, sem_g, sem_o):
        wid = lax.axis_index("s") * NC + lax.axis_index("c")
        base = wid * b_per_w

        pltpu.sync_copy(idx_hbm.at[pl.ds(base, b_per_w)], idx_v)

        def gather_copy(i, b):
            return pltpu.make_async_copy(
                table_hbm.at[idx_v.at[pl.ds(i * C, C)]], rows_v.at[b], sem_g
            )

        def out_copy(i, b):
            return pltpu.make_async_copy(
                rows_v.at[b], out_hbm.at[pl.ds(base + i * C, C)], sem_o
            )

        gather_copy(0, 0).start()

        def body(g, carry):
            for b in range(_NBUF):
                i = g * _NBUF + b
                b2 = (b + 1) % _NBUF
                if b == _NBUF - 1:

                    @pl.when(g < n_groups - 1)
                    def _():
                        out_copy(i + 1 - _NBUF, b2).wait()
                        gather_copy(i + 1, b2).start()

                else:

                    @pl.when(g >= 1)
                    def _():
                        out_copy(i + 1 - _NBUF, b2).wait()

                    gather_copy(i + 1, b2).start()

                gather_copy(i, b).wait()
                out_copy(i, b).start()
            return carry

        lax.fori_loop(0, n_groups, body, 0)

        for k in range(_NBUF):
            j = n_chunks - _NBUF + k
            out_copy(j, j % _NBUF).wait()

    return gather_kernel


def kernel(inputs, table):
    B, S = inputs.shape
    V, D = table.shape
    idx = inputs.reshape(-1).astype(jnp.int32)
    gather = _make_gather(V, D, B * S)
    out = gather(table, idx)
    return out.reshape(B, S, D)


# speedup vs baseline: 4.8700x; 1.4717x over previous
import functools

import jax
import jax.numpy as jnp
from jax import lax
from jax.experimental import pallas as pl
from jax.experimental.pallas import tpu as pltpu
from jax.experimental.pallas import tpu_sc as plsc

_NBUF = 2
_CHUNK = 800


def _make_gather(V, D, N):
    info = plsc.get_sparse_core_info()
    NC, NS = info.num_cores, info.num_subcores
    NW = NC * NS
    assert N % NW == 0
    b_per_w = N // NW
    C = _CHUNK
    assert b_per_w % (C * _NBUF) == 0
    n_chunks = b_per_w // C
    n_groups = n_chunks // _NBUF

    mesh = plsc.VectorSubcoreMesh(core_axis_name="c", subcore_axis_name="s")

    @functools.partial(
        pl.kernel,
        mesh=mesh,
        out_type=jax.ShapeDtypeStruct((N, 128), jnp.float32),
        compiler_params=pltpu.CompilerParams(use_tc_tiling_on_sc=False),
        scratch_types=[
            pltpu.VMEM((b_per_w,), jnp.int32),
            pltpu.VMEM((_NBUF, C, D), jnp.float32),
            pltpu.SemaphoreType.DMA,
            pltpu.SemaphoreType.DMA,
        ],
    )
    def gather_kernel(table_hbm, idx_hbm, out_hbm, idx_v, rows_v, sem_g, sem_o):
        wid = lax.axis_index("s") * NC + lax.axis_index("c")
        base = wid * b_per_w

        pltpu.sync_copy(idx_hbm.at[pl.ds(base, b_per_w)], idx_v)

        def gather_copy(i, b):
            return pltpu.make_async_copy(
                table_hbm.at[idx_v.at[pl.ds(i * C, C)]], rows_v.at[b], sem_g
            )

        def out_copy(i, b):
            return pltpu.make_async_copy(
                rows_v.at[b], out_hbm.at[pl.ds(base + i * C, C), pl.ds(0, D)], sem_o
            )

        gather_copy(0, 0).start()

        def body(g, carry):
            for b in range(_NBUF):
                i = g * _NBUF + b
                b2 = (b + 1) % _NBUF
                if b == _NBUF - 1:

                    @pl.when(g < n_groups - 1)
                    def _():
                        out_copy(i + 1 - _NBUF, b2).wait()
                        gather_copy(i + 1, b2).start()

                else:

                    @pl.when(g >= 1)
                    def _():
                        out_copy(i + 1 - _NBUF, b2).wait()

                    gather_copy(i + 1, b2).start()

                gather_copy(i, b).wait()
                out_copy(i, b).start()
            return carry

        lax.fori_loop(0, n_groups, body, 0)

        for k in range(_NBUF):
            j = n_chunks - _NBUF + k
            out_copy(j, j % _NBUF).wait()

    return gather_kernel


def kernel(inputs, table):
    B, S = inputs.shape
    V, D = table.shape
    idx = inputs.reshape(-1).astype(jnp.int32)
    gather = _make_gather(V, D, B * S)
    out = gather(table, idx)
    return out[:, :D].reshape(B, S, D)
